# baseline (speedup 1.0000x reference)
import functools

import jax
import jax.numpy as jnp
from jax import lax
from jax.experimental import pallas as pl
from jax.experimental.pallas import tpu as pltpu
from jax.experimental.pallas import tpu_sc as plsc

N_BINS = 64
NC, NS, L = 2, 16, 16
NW = NC * NS


def _sc_hist_t(xt, params, B, C):
    rows_per_w = B // NW
    chunk = 128
    n_chunks = rows_per_w // chunk
    table_len = C * N_BINS

    mesh = plsc.VectorSubcoreMesh(
        core_axis_name="c", subcore_axis_name="s", num_cores=NC,
        num_subcores=NS)

    @functools.partial(
        pl.kernel,
        out_type=jax.ShapeDtypeStruct((NW, table_len), jnp.float32),
        mesh=mesh,
        scratch_types=[
            pltpu.VMEM((C, chunk), jnp.float32),
            pltpu.VMEM((C, chunk), jnp.float32),
            pltpu.VMEM((3 * C * L,), jnp.float32),
            pltpu.VMEM((table_len,), jnp.float32),
            pltpu.SemaphoreType.DMA,
            pltpu.SemaphoreType.DMA,
        ],
        compiler_params=pltpu.CompilerParams(needs_layout_passes=False),
    )
    def hist_kernel(xt_hbm, p_hbm, out_hbm, x_buf0, x_buf1, pbuf, table,
                    sem0, sem1):
        wid = lax.axis_index("s") * NC + lax.axis_index("c")
        base_b = wid * rows_per_w
        bufs = (x_buf0, x_buf1)
        sems = (sem0, sem1)

        copies = [None] * n_chunks
        copies[0] = pltpu.async_copy(
            xt_hbm.at[:, pl.ds(base_b, chunk)], x_buf0, sem0)

        pltpu.sync_copy(p_hbm, pbuf)

        zeros = jnp.zeros((L,), jnp.float32)

        @plsc.parallel_loop(0, table_len // L, unroll=8)
        def zero_body(i):
            table[pl.ds(pl.multiple_of(i * L, L), L)] = zeros

        ones = jnp.ones((L,), jnp.float32)

        for k in range(n_chunks):
            if k + 1 < n_chunks:
                copies[k + 1] = pltpu.async_copy(
                    xt_hbm.at[:, pl.ds(base_b + (k + 1) * chunk, chunk)],
                    bufs[(k + 1) % 2], sems[(k + 1) % 2])
            copies[k].wait()
            x_buf = bufs[k % 2]

            @plsc.parallel_loop(0, C, unroll=2)
            def col_body(c):
                co = pl.multiple_of(c * L, L)
                av = pbuf[pl.ds(co, L)]
                bv = pbuf[pl.ds(C * L + co, L)]
                sv = pbuf[pl.ds(2 * C * L + co, L)]
                base = jnp.full((L,), c, jnp.int32) * N_BINS
                for i in range(chunk // L):
                    xv = x_buf[c, pl.ds(i * L, L)]
                    u = (xv - av) * sv
                    cl = jnp.minimum(jnp.maximum(u, 0.0), float(N_BINS - 1))
                    idx = cl.astype(jnp.int32) + base
                    valid = (xv >= av) & (xv <= bv)
                    plsc.addupdate_scatter(table, [idx], ones, mask=valid)

        pltpu.sync_copy(table, out_hbm.at[wid])

    return hist_kernel(xt, params)


def _sc_hist(x2d, ab, B, C):
    rows_per_w = B // NW
    n_chunks = 4
    chunk = rows_per_w // n_chunks
    n_groups = (C + L - 1) // L
    table_len = C * N_BINS

    mesh = plsc.VectorSubcoreMesh(
        core_axis_name="c", subcore_axis_name="s", num_cores=NC,
        num_subcores=NS)

    @functools.partial(
        pl.kernel,
        out_type=jax.ShapeDtypeStruct((NW, table_len), jnp.float32),
        mesh=mesh,
        scratch_types=[
            pltpu.VMEM((chunk, C), jnp.float32),
            pltpu.VMEM((chunk, C), jnp.float32),
            pltpu.VMEM((2 * C,), jnp.float32),
            pltpu.VMEM((table_len,), jnp.float32),
            pltpu.SemaphoreType.DMA,
            pltpu.SemaphoreType.DMA,
        ],
        compiler_params=pltpu.CompilerParams(needs_layout_passes=False),
    )
    def hist_kernel(x_hbm, ab_hbm, out_hbm, x_buf0, x_buf1, ab_buf, table,
                    sem0, sem1):
        wid = lax.axis_index("s") * NC + lax.axis_index("c")
        base_row = wid * rows_per_w
        bufs = (x_buf0, x_buf1)
        sems = (sem0, sem1)

        copies = [None] * n_chunks
        copies[0] = pltpu.async_copy(
            x_hbm.at[pl.ds(base_row, chunk), :], x_buf0, sem0)

        pltpu.sync_copy(ab_hbm, ab_buf)

        zeros = jnp.zeros((L,), jnp.float32)

        @plsc.parallel_loop(0, table_len // L, unroll=8)
        def zero_body(i):
            table[pl.ds(pl.multiple_of(i * L, L), L)] = zeros

        lane = lax.iota(jnp.int32, L)
        ones = jnp.ones((L,), jnp.float32)

        for k in range(n_chunks):
            if k + 1 < n_chunks:
                copies[k + 1] = pltpu.async_copy(
                    x_hbm.at[pl.ds(base_row + (k + 1) * chunk, chunk), :],
                    bufs[(k + 1) % 2], sems[(k + 1) % 2])
            copies[k].wait()
            x_buf = bufs[k % 2]

            for g in range(n_groups):
                off = C - L if g == n_groups - 1 else g * L
                full = (g + 1) * L <= C
                av = ab_buf[pl.ds(off, L)]
                bv = ab_buf[pl.ds(C + off, L)]
                sv = float(N_BINS) / (bv - av)
                base = (lane + off) * N_BINS
                gmask = None if full else lane >= (g * L - off)

                @plsc.parallel_loop(0, chunk, unroll=8)
                def row_body(r):
                    xv = x_buf[r, pl.ds(off, L)]
                    u = (xv - av) * sv
                    cl = jnp.minimum(jnp.maximum(u, 0.0), float(N_BINS - 1))
                    idx = cl.astype(jnp.int32) + base
                    valid = (xv >= av) & (xv <= bv)
                    if gmask is not None:
                        valid = valid & gmask
                    plsc.addupdate_scatter(table, [idx], ones, mask=valid)

        pltpu.sync_copy(table, out_hbm.at[wid])

    return hist_kernel(x2d, ab)


def _tc_loss(tables3, dens3, B, n_entries):
    def loss_body(tabs_ref, dens_ref, out_ref):
        counts = jnp.sum(tabs_ref[...], axis=0)
        diff = jnp.abs(counts * (1.0 / B) - dens_ref[...])
        out_ref[0, 0] = jnp.sum(diff) * (1.0 / n_entries)

    out = pl.pallas_call(
        loss_body,
        out_shape=jax.ShapeDtypeStruct((1, 1), jnp.float32),
        out_specs=pl.BlockSpec(memory_space=pltpu.SMEM),
    )(tables3, dens3)
    return out[0, 0]


def kernel(x_fake, densities, bin_min, bin_max):
    B, T, D = x_fake.shape
    C = T * D
    pvals = jnp.stack(
        [bin_min, bin_max, jnp.float32(N_BINS) / (bin_max - bin_min)])
    params = jnp.broadcast_to(pvals[:, :, None], (3, C, L)).reshape(3 * C * L)
    n_split = 4
    part = B // n_split
    tabs = [
        _sc_hist_t(
            x_fake[i * part:(i + 1) * part].reshape(part, C).T, params,
            part, C)
        for i in range(n_split)
    ]
    tables3 = jnp.concatenate(tabs).reshape(
        n_split * NW, C * N_BINS // 128, 128)
    dens3 = densities.reshape(C * N_BINS // 128, 128)
    return _tc_loss(tables3, dens3, B, C * N_BINS)


# speedup vs baseline: 1.2876x; 1.2876x over previous
import functools

import jax
import jax.numpy as jnp
from jax import lax
from jax.experimental import pallas as pl
from jax.experimental.pallas import tpu as pltpu
from jax.experimental.pallas import tpu_sc as plsc

N_BINS = 64
NC, NS, L = 2, 16, 16
NW = NC * NS


def _sc_hist_t(xt, params, B, C):
    rows_per_w = B // NW
    chunk = 128
    n_chunks = rows_per_w // chunk
    table_len = C * N_BINS

    mesh = plsc.VectorSubcoreMesh(
        core_axis_name="c", subcore_axis_name="s", num_cores=NC,
        num_subcores=NS)

    @functools.partial(
        pl.kernel,
        out_type=jax.ShapeDtypeStruct((NW, table_len), jnp.float32),
        mesh=mesh,
        scratch_types=[
            pltpu.VMEM((C, chunk), jnp.float32),
            pltpu.VMEM((C, chunk), jnp.float32),
            pltpu.VMEM((3 * C * L,), jnp.float32),
            pltpu.VMEM((table_len,), jnp.float32),
            pltpu.SemaphoreType.DMA,
            pltpu.SemaphoreType.DMA,
        ],
        compiler_params=pltpu.CompilerParams(needs_layout_passes=False),
    )
    def hist_kernel(xt_hbm, p_hbm, out_hbm, x_buf0, x_buf1, pbuf, table,
                    sem0, sem1):
        wid = lax.axis_index("s") * NC + lax.axis_index("c")
        base_b = wid * rows_per_w
        bufs = (x_buf0, x_buf1)
        sems = (sem0, sem1)

        copies = [None] * n_chunks
        copies[0] = pltpu.async_copy(
            xt_hbm.at[:, pl.ds(base_b, chunk)], x_buf0, sem0)

        pltpu.sync_copy(p_hbm, pbuf)

        zeros = jnp.zeros((L,), jnp.float32)

        @plsc.parallel_loop(0, table_len // L, unroll=8)
        def zero_body(i):
            table[pl.ds(pl.multiple_of(i * L, L), L)] = zeros

        ones = jnp.ones((L,), jnp.float32)

        for k in range(n_chunks):
            if k + 1 < n_chunks:
                copies[k + 1] = pltpu.async_copy(
                    xt_hbm.at[:, pl.ds(base_b + (k + 1) * chunk, chunk)],
                    bufs[(k + 1) % 2], sems[(k + 1) % 2])
            copies[k].wait()
            x_buf = bufs[k % 2]

            @plsc.parallel_loop(0, C, unroll=2)
            def col_body(c):
                co = pl.multiple_of(c * L, L)
                av = pbuf[pl.ds(co, L)]
                bv = pbuf[pl.ds(C * L + co, L)]
                sv = pbuf[pl.ds(2 * C * L + co, L)]
                base = jnp.full((L,), c, jnp.int32) * N_BINS
                for i in range(chunk // L):
                    xv = x_buf[c, pl.ds(i * L, L)]
                    u = (xv - av) * sv
                    cl = jnp.minimum(jnp.maximum(u, 0.0), float(N_BINS - 1))
                    idx = cl.astype(jnp.int32) + base
                    valid = (xv >= av) & (xv <= bv)
                    plsc.addupdate_scatter(table, [idx], ones, mask=valid)

        pltpu.sync_copy(table, out_hbm.at[wid])

    return hist_kernel(xt, params)


def _sc_hist(x2d, ab, B, C):
    rows_per_w = B // NW
    n_chunks = 4
    chunk = rows_per_w // n_chunks
    n_groups = (C + L - 1) // L
    table_len = C * N_BINS

    mesh = plsc.VectorSubcoreMesh(
        core_axis_name="c", subcore_axis_name="s", num_cores=NC,
        num_subcores=NS)

    @functools.partial(
        pl.kernel,
        out_type=jax.ShapeDtypeStruct((NW, table_len), jnp.float32),
        mesh=mesh,
        scratch_types=[
            pltpu.VMEM((chunk, C), jnp.float32),
            pltpu.VMEM((chunk, C), jnp.float32),
            pltpu.VMEM((2 * C,), jnp.float32),
            pltpu.VMEM((table_len,), jnp.float32),
            pltpu.SemaphoreType.DMA,
            pltpu.SemaphoreType.DMA,
        ],
        compiler_params=pltpu.CompilerParams(needs_layout_passes=False),
    )
    def hist_kernel(x_hbm, ab_hbm, out_hbm, x_buf0, x_buf1, ab_buf, table,
                    sem0, sem1):
        wid = lax.axis_index("s") * NC + lax.axis_index("c")
        base_row = wid * rows_per_w
        bufs = (x_buf0, x_buf1)
        sems = (sem0, sem1)

        copies = [None] * n_chunks
        copies[0] = pltpu.async_copy(
            x_hbm.at[pl.ds(base_row, chunk), :], x_buf0, sem0)

        pltpu.sync_copy(ab_hbm, ab_buf)

        zeros = jnp.zeros((L,), jnp.float32)

        @plsc.parallel_loop(0, table_len // L, unroll=8)
        def zero_body(i):
            table[pl.ds(pl.multiple_of(i * L, L), L)] = zeros

        lane = lax.iota(jnp.int32, L)
        ones = jnp.ones((L,), jnp.float32)

        for k in range(n_chunks):
            if k + 1 < n_chunks:
                copies[k + 1] = pltpu.async_copy(
                    x_hbm.at[pl.ds(base_row + (k + 1) * chunk, chunk), :],
                    bufs[(k + 1) % 2], sems[(k + 1) % 2])
            copies[k].wait()
            x_buf = bufs[k % 2]

            for g in range(n_groups):
                off = C - L if g == n_groups - 1 else g * L
                full = (g + 1) * L <= C
                av = ab_buf[pl.ds(off, L)]
                bv = ab_buf[pl.ds(C + off, L)]
                sv = float(N_BINS) / (bv - av)
                base = (lane + off) * N_BINS
                gmask = None if full else lane >= (g * L - off)

                @plsc.parallel_loop(0, chunk, unroll=8)
                def row_body(r):
                    xv = x_buf[r, pl.ds(off, L)]
                    u = (xv - av) * sv
                    cl = jnp.minimum(jnp.maximum(u, 0.0), float(N_BINS - 1))
                    idx = cl.astype(jnp.int32) + base
                    valid = (xv >= av) & (xv <= bv)
                    if gmask is not None:
                        valid = valid & gmask
                    plsc.addupdate_scatter(table, [idx], ones, mask=valid)

        pltpu.sync_copy(table, out_hbm.at[wid])

    return hist_kernel(x2d, ab)


def _tc_loss(tables3, dens3, B, n_entries):
    def loss_body(tabs_ref, dens_ref, out_ref):
        counts = jnp.sum(tabs_ref[...], axis=0)
        diff = jnp.abs(counts * (1.0 / B) - dens_ref[...])
        out_ref[0, 0] = jnp.sum(diff) * (1.0 / n_entries)

    out = pl.pallas_call(
        loss_body,
        out_shape=jax.ShapeDtypeStruct((1, 1), jnp.float32),
        out_specs=pl.BlockSpec(memory_space=pltpu.SMEM),
    )(tables3, dens3)
    return out[0, 0]


def kernel(x_fake, densities, bin_min, bin_max):
    B, T, D = x_fake.shape
    C = T * D
    pvals = jnp.stack(
        [bin_min, bin_max, jnp.float32(N_BINS) / (bin_max - bin_min)])
    params = jnp.broadcast_to(pvals[:, :, None], (3, C, L)).reshape(3 * C * L)
    n_split = 2
    part = B // n_split
    tabs = [
        _sc_hist_t(
            x_fake[i * part:(i + 1) * part].reshape(part, C).T, params,
            part, C)
        for i in range(n_split)
    ]
    tables3 = jnp.concatenate(tabs).reshape(
        n_split * NW, C * N_BINS // 128, 128)
    dens3 = densities.reshape(C * N_BINS // 128, 128)
    return _tc_loss(tables3, dens3, B, C * N_BINS)
